# baseline (speedup 1.0000x reference)
import functools

import jax
import jax.numpy as jnp
from jax import lax
from jax.experimental import pallas as pl
from jax.experimental.pallas import tpu as pltpu
from jax.experimental.pallas import tpu_sc as plsc

B = 4096
L = 50
D = 64
NC, NS = 2, 16
NW = NC * NS
N = B * L
PER_W = N // NW

PCHUNK = 64
NPCHUNK = PER_W // PCHUNK
PRING = 4
NITER = NPCHUNK // PRING

RCHUNK = 64
NRCHUNK = PER_W // RCHUNK
RGROUPS = RCHUNK // 16
RRING = 4
R_PER_ITER = NRCHUNK // NITER

REL_V = 1000

_GDN = lax.GatherDimensionNumbers(offset_dims=(), collapsed_slice_dims=(0,),
                                  start_index_map=(0,))


def _bcast_lane(vec, l):
    idx = jnp.full((16, 1), l, dtype=jnp.int32)
    return lax.gather(vec, idx, dimension_numbers=_GDN, slice_sizes=(1,),
                      mode=lax.GatherScatterMode.PROMISE_IN_BOUNDS)


def _body(pos_idx_hbm, rel_idx_hbm, pos_tab_hbm, rel_tab_hbm,
          pos_out_hbm, rel_out_hbm,
          pidx_v, ridx_v, rtab_v, pbuf, rbuf,
          gsems, wsems, rwsem):
    wid = lax.axis_index("s") * NC + lax.axis_index("c")
    base = wid * PER_W
    lane = jnp.arange(16, dtype=jnp.int32)

    pltpu.sync_copy(pos_idx_hbm.at[wid], pidx_v)
    pltpu.sync_copy(rel_idx_hbm.at[wid], ridx_v)
    pltpu.sync_copy(rel_tab_hbm, rtab_v)

    def fire_pos_gather(j, r):
        pltpu.async_copy(pos_tab_hbm.at[pidx_v.at[j]], pbuf.at[r],
                         gsems[r])

    def drain_pos_gather(j, r):
        pltpu.make_async_copy(pos_tab_hbm.at[pidx_v.at[j]], pbuf.at[r],
                              gsems[r]).wait()

    def fire_pos_write(j, r):
        pltpu.async_copy(pbuf.at[r],
                         pos_out_hbm.at[pl.ds(base + j * PCHUNK, PCHUNK)],
                         wsems[r])

    def drain_pos_write(j, r):
        pltpu.make_async_copy(pbuf.at[r],
                              pos_out_hbm.at[pl.ds(base + j * PCHUNK, PCHUNK)],
                              wsems[r]).wait()

    def rel_write_desc(j, p):
        return pltpu.make_async_copy(
            rbuf.at[p],
            rel_out_hbm.at[pl.ds(base + j * RCHUNK, RCHUNK)],
            rwsem)

    offs = [jnp.full((16,), k * 16, jnp.int32) + lane for k in range(D // 16)]

    def rel_compute_chunk(j):
        p = lax.rem(j, RRING)

        @pl.when(j >= RRING)
        def _():
            rel_write_desc(j - RRING, p).wait()

        def group_step(g, carry):
            vidx = ridx_v[pl.ds(j * RCHUNK + g * 16, 16)]
            bases = vidx * D
            for l in range(16):
                bl = _bcast_lane(bases, l)
                for k in range(D // 16):
                    vec = plsc.load_gather(rtab_v, [bl + offs[k]])
                    rbuf[p, g * 16 + l, pl.ds(k * 16, 16)] = vec
            return carry

        lax.fori_loop(0, RGROUPS, group_step, 0)
        pltpu.async_copy(rbuf.at[p],
                         rel_out_hbm.at[pl.ds(base + j * RCHUNK, RCHUNK)],
                         rwsem)

    def body(i, carry):
        for r in range(PRING):
            @pl.when(i > 0)
            def _(r=r):
                drain_pos_write(PRING * (i - 1) + r, r)
            fire_pos_gather(PRING * i + r, r)

        for k in range(R_PER_ITER):
            rel_compute_chunk(R_PER_ITER * i + k)

        for r in range(PRING):
            drain_pos_gather(PRING * i + r, r)
            fire_pos_write(PRING * i + r, r)
        return carry

    lax.fori_loop(0, NITER, body, 0)

    for r in range(PRING):
        drain_pos_write(PRING * (NITER - 1) + r, r)
    for p in range(RRING):
        rel_write_desc(NRCHUNK - RRING + p, p).wait()


@jax.jit
def _tree_embedding(position_idx, rel_idx, position_table, relation_table):
    pos_idx = position_idx.reshape(NW, NPCHUNK, PCHUNK).astype(jnp.int32)
    ridx = rel_idx.reshape(NW, PER_W).astype(jnp.int32)

    mesh = plsc.VectorSubcoreMesh(core_axis_name="c", subcore_axis_name="s")
    kern = pl.kernel(
        _body,
        out_type=(
            jax.ShapeDtypeStruct((N, D), jnp.float32),
            jax.ShapeDtypeStruct((N, D), jnp.float32),
        ),
        mesh=mesh,
        scratch_types=[
            pltpu.VMEM((NPCHUNK, PCHUNK), jnp.int32),
            pltpu.VMEM((PER_W,), jnp.int32),
            pltpu.VMEM((REL_V * D,), jnp.float32),
            pltpu.VMEM((PRING, PCHUNK, D), jnp.float32),
            pltpu.VMEM((RRING, RCHUNK, D), jnp.float32),
            [pltpu.SemaphoreType.DMA] * PRING,
            [pltpu.SemaphoreType.DMA] * PRING,
            pltpu.SemaphoreType.DMA,
        ],
        compiler_params=pltpu.CompilerParams(use_tc_tiling_on_sc=False,
                                             needs_layout_passes=False),
    )
    pos_out, rel_out = kern(pos_idx, ridx, position_table,
                            relation_table.reshape(REL_V * D))
    return (rel_out.reshape(B, L, D), pos_out.reshape(B, L, D))


def kernel(position_idx, rel_idx, position_table, relation_table):
    return _tree_embedding(position_idx, rel_idx, position_table,
                           relation_table)


# speedup vs baseline: 1.0077x; 1.0077x over previous
import functools

import jax
import jax.numpy as jnp
from jax import lax
from jax.experimental import pallas as pl
from jax.experimental.pallas import tpu as pltpu
from jax.experimental.pallas import tpu_sc as plsc

B = 4096
L = 50
D = 64
NC, NS = 2, 16
NW = NC * NS
N = B * L
PER_W = N // NW

PCHUNK = 64
NPCHUNK = PER_W // PCHUNK
PRING = 4
NITER = NPCHUNK // PRING

RCHUNK = 64
NRCHUNK = PER_W // RCHUNK
RGROUPS = RCHUNK // 16
RRING = 4
R_PER_ITER = NRCHUNK // NITER

REL_V = 1000

_GDN = lax.GatherDimensionNumbers(offset_dims=(), collapsed_slice_dims=(0,),
                                  start_index_map=(0,))


def _bcast_lane(vec, l):
    idx = jnp.full((16, 1), l, dtype=jnp.int32)
    return lax.gather(vec, idx, dimension_numbers=_GDN, slice_sizes=(1,),
                      mode=lax.GatherScatterMode.PROMISE_IN_BOUNDS)


def _body(pos_idx_hbm, rel_idx_hbm, pos_tab_hbm, rel_tab_hbm,
          pos_out_hbm, rel_out_hbm,
          pidx_v, ridx_v, pbuf, rbuf,
          gsems, wsems, rgsems, rwsem):
    wid = lax.axis_index("s") * NC + lax.axis_index("c")
    base = wid * PER_W
    lane = jnp.arange(16, dtype=jnp.int32)

    pltpu.sync_copy(pos_idx_hbm.at[wid], pidx_v)
    pltpu.sync_copy(rel_idx_hbm.at[wid], ridx_v)

    def fire_pos_gather(j, r):
        pltpu.async_copy(pos_tab_hbm.at[pidx_v.at[j]], pbuf.at[r],
                         gsems[r])

    def drain_pos_gather(j, r):
        pltpu.make_async_copy(pos_tab_hbm.at[pidx_v.at[j]], pbuf.at[r],
                              gsems[r]).wait()

    def fire_pos_write(j, r):
        pltpu.async_copy(pbuf.at[r],
                         pos_out_hbm.at[pl.ds(base + j * PCHUNK, PCHUNK)],
                         wsems[r])

    def drain_pos_write(j, r):
        pltpu.make_async_copy(pbuf.at[r],
                              pos_out_hbm.at[pl.ds(base + j * PCHUNK, PCHUNK)],
                              wsems[r]).wait()

    def rel_write_desc(j, p):
        return pltpu.make_async_copy(
            rbuf.at[p],
            rel_out_hbm.at[pl.ds(base + j * RCHUNK, RCHUNK)],
            rwsem)

    def fire_rel_gather(j, p):
        pltpu.async_copy(rel_tab_hbm.at[ridx_v.at[j]], rbuf.at[p], rgsems[p])

    def drain_rel_gather(j, p):
        pltpu.make_async_copy(rel_tab_hbm.at[ridx_v.at[j]], rbuf.at[p],
                              rgsems[p]).wait()

    def rel_engine_chunk(j, p):
        @pl.when(j >= RRING)
        def _():
            rel_write_desc(j - RRING, p).wait()
        fire_rel_gather(j, p)

    def rel_engine_finish(j, p):
        drain_rel_gather(j, p)
        pltpu.async_copy(rbuf.at[p],
                         rel_out_hbm.at[pl.ds(base + j * RCHUNK, RCHUNK)],
                         rwsem)

    def body(i, carry):
        for r in range(PRING):
            @pl.when(i > 0)
            def _(r=r):
                drain_pos_write(PRING * (i - 1) + r, r)
            fire_pos_gather(PRING * i + r, r)

        for k in range(R_PER_ITER):
            rel_engine_chunk(R_PER_ITER * i + k, k)

        for r in range(PRING):
            drain_pos_gather(PRING * i + r, r)
            fire_pos_write(PRING * i + r, r)
        for k in range(R_PER_ITER):
            rel_engine_finish(R_PER_ITER * i + k, k)
        return carry

    lax.fori_loop(0, NITER, body, 0)

    for r in range(PRING):
        drain_pos_write(PRING * (NITER - 1) + r, r)
    for p in range(RRING):
        rel_write_desc(NRCHUNK - RRING + p, p).wait()


@jax.jit
def _tree_embedding(position_idx, rel_idx, position_table, relation_table):
    pos_idx = position_idx.reshape(NW, NPCHUNK, PCHUNK).astype(jnp.int32)
    ridx = rel_idx.reshape(NW, NRCHUNK, RCHUNK).astype(jnp.int32)

    mesh = plsc.VectorSubcoreMesh(core_axis_name="c", subcore_axis_name="s")
    kern = pl.kernel(
        _body,
        out_type=(
            jax.ShapeDtypeStruct((N, D), jnp.float32),
            jax.ShapeDtypeStruct((N, D), jnp.float32),
        ),
        mesh=mesh,
        scratch_types=[
            pltpu.VMEM((NPCHUNK, PCHUNK), jnp.int32),
            pltpu.VMEM((NRCHUNK, RCHUNK), jnp.int32),
            pltpu.VMEM((PRING, PCHUNK, D), jnp.float32),
            pltpu.VMEM((RRING, RCHUNK, D), jnp.float32),
            [pltpu.SemaphoreType.DMA] * PRING,
            [pltpu.SemaphoreType.DMA] * PRING,
            [pltpu.SemaphoreType.DMA] * RRING,
            pltpu.SemaphoreType.DMA,
        ],
        compiler_params=pltpu.CompilerParams(use_tc_tiling_on_sc=False,
                                             needs_layout_passes=False),
    )
    pos_out, rel_out = kern(pos_idx, ridx, position_table,
                            relation_table)
    return (rel_out.reshape(B, L, D), pos_out.reshape(B, L, D))


def kernel(position_idx, rel_idx, position_table, relation_table):
    return _tree_embedding(position_idx, rel_idx, position_table,
                           relation_table)


# speedup vs baseline: 1.0080x; 1.0003x over previous
import jax
import jax.numpy as jnp
from jax import lax
from jax.experimental import pallas as pl
from jax.experimental.pallas import tpu as pltpu
from jax.experimental.pallas import tpu_sc as plsc

B = 4096
L = 50
D = 64
NC, NS = 2, 16
NW = NC * NS
N = B * L
PER_W = N // NW

PCHUNK = 64
NPCHUNK = PER_W // PCHUNK
PRING = 4
NITER = NPCHUNK // PRING

RCHUNK = 64
NRCHUNK = PER_W // RCHUNK
RRING = 4
R_PER_ITER = NRCHUNK // NITER

REL_V = 1000

def _body(pos_idx_hbm, rel_idx_hbm, pos_tab_hbm, rel_tab_hbm,
          pos_out_hbm, rel_out_hbm,
          pidx_v, ridx_v, pbuf, rbuf,
          gsems, wsems, rgsems, rwsem):
    wid = lax.axis_index("s") * NC + lax.axis_index("c")
    base = wid * PER_W
    lane = jnp.arange(16, dtype=jnp.int32)

    pltpu.sync_copy(pos_idx_hbm.at[wid], pidx_v)
    pltpu.sync_copy(rel_idx_hbm.at[wid], ridx_v)

    def fire_pos_gather(j, r):
        pltpu.async_copy(pos_tab_hbm.at[pidx_v.at[j]], pbuf.at[r],
                         gsems[r])

    def drain_pos_gather(j, r):
        pltpu.make_async_copy(pos_tab_hbm.at[pidx_v.at[j]], pbuf.at[r],
                              gsems[r]).wait()

    def fire_pos_write(j, r):
        pltpu.async_copy(pbuf.at[r],
                         pos_out_hbm.at[pl.ds(base + j * PCHUNK, PCHUNK)],
                         wsems[r])

    def drain_pos_write(j, r):
        pltpu.make_async_copy(pbuf.at[r],
                              pos_out_hbm.at[pl.ds(base + j * PCHUNK, PCHUNK)],
                              wsems[r]).wait()

    def rel_write_desc(j, p):
        return pltpu.make_async_copy(
            rbuf.at[p],
            rel_out_hbm.at[pl.ds(base + j * RCHUNK, RCHUNK)],
            rwsem)

    def fire_rel_gather(j, p):
        pltpu.async_copy(rel_tab_hbm.at[ridx_v.at[j]], rbuf.at[p], rgsems[p])

    def drain_rel_gather(j, p):
        pltpu.make_async_copy(rel_tab_hbm.at[ridx_v.at[j]], rbuf.at[p],
                              rgsems[p]).wait()

    def rel_engine_chunk(j, p):
        @pl.when(j >= RRING)
        def _():
            rel_write_desc(j - RRING, p).wait()
        fire_rel_gather(j, p)

    def rel_engine_finish(j, p):
        drain_rel_gather(j, p)
        pltpu.async_copy(rbuf.at[p],
                         rel_out_hbm.at[pl.ds(base + j * RCHUNK, RCHUNK)],
                         rwsem)

    def body(i, carry):
        for r in range(PRING):
            @pl.when(i > 0)
            def _(r=r):
                drain_pos_write(PRING * (i - 1) + r, r)
            fire_pos_gather(PRING * i + r, r)

        for k in range(R_PER_ITER):
            rel_engine_chunk(R_PER_ITER * i + k, k)

        for r in range(PRING):
            drain_pos_gather(PRING * i + r, r)
            fire_pos_write(PRING * i + r, r)
        for k in range(R_PER_ITER):
            rel_engine_finish(R_PER_ITER * i + k, k)
        return carry

    lax.fori_loop(0, NITER, body, 0)

    for r in range(PRING):
        drain_pos_write(PRING * (NITER - 1) + r, r)
    for p in range(RRING):
        rel_write_desc(NRCHUNK - RRING + p, p).wait()


@jax.jit
def _tree_embedding(position_idx, rel_idx, position_table, relation_table):
    pos_idx = position_idx.reshape(NW, NPCHUNK, PCHUNK).astype(jnp.int32)
    ridx = rel_idx.reshape(NW, NRCHUNK, RCHUNK).astype(jnp.int32)

    mesh = plsc.VectorSubcoreMesh(core_axis_name="c", subcore_axis_name="s")
    kern = pl.kernel(
        _body,
        out_type=(
            jax.ShapeDtypeStruct((N, D), jnp.float32),
            jax.ShapeDtypeStruct((N, D), jnp.float32),
        ),
        mesh=mesh,
        scratch_types=[
            pltpu.VMEM((NPCHUNK, PCHUNK), jnp.int32),
            pltpu.VMEM((NRCHUNK, RCHUNK), jnp.int32),
            pltpu.VMEM((PRING, PCHUNK, D), jnp.float32),
            pltpu.VMEM((RRING, RCHUNK, D), jnp.float32),
            [pltpu.SemaphoreType.DMA] * PRING,
            [pltpu.SemaphoreType.DMA] * PRING,
            [pltpu.SemaphoreType.DMA] * RRING,
            pltpu.SemaphoreType.DMA,
        ],
        compiler_params=pltpu.CompilerParams(use_tc_tiling_on_sc=False,
                                             needs_layout_passes=False),
    )
    pos_out, rel_out = kern(pos_idx, ridx, position_table,
                            relation_table)
    return (rel_out.reshape(B, L, D), pos_out.reshape(B, L, D))


def kernel(position_idx, rel_idx, position_table, relation_table):
    return _tree_embedding(position_idx, rel_idx, position_table,
                           relation_table)
